# baseline (speedup 1.0000x reference)
import jax
import jax.numpy as jnp
from jax import lax
from jax.experimental import pallas as pl
from jax.experimental.pallas import tpu as pltpu
from jax.experimental.pallas import tpu_sc as plsc

N = 50000
FEATS = 256
K = 2000
NPAD = 51200
RB = 2048
GRID1 = NPAD // RB
INT_MIN = -(2**31)

NT = 16
CH = NPAD // NT
NCHUNK = CH // 16
KP = 2048
CAP_L = 256
CAP_G = 2816
RPT = CAP_G // NT
GPT = KP // NT


def _keys_body(x_ref, w_ref, m_ref, s_ref, o_ref):
    x = x_ref[...]
    mv = jnp.dot(x, w_ref[...])
    sc = mv / s_ref[0, 0] + m_ref[...]
    b = lax.bitcast_convert_type(sc, jnp.int32)
    key = jnp.where(b < 0, b ^ jnp.int32(0x7FFFFFFF), b)
    pos = pl.program_id(0) * RB + lax.broadcasted_iota(jnp.int32, (RB, 1), 0)
    key = jnp.where(pos < N, key, jnp.int32(INT_MIN))
    o_ref[...] = key.reshape(RB // 128, 128)


def _scores_to_keys(node_embs, mask, scorer, s2d):
    return pl.pallas_call(
        _keys_body,
        grid=(GRID1,),
        in_specs=[
            pl.BlockSpec((RB, FEATS), lambda i: (i, 0)),
            pl.BlockSpec((FEATS, 1), lambda i: (0, 0)),
            pl.BlockSpec((RB, 1), lambda i: (i, 0)),
            pl.BlockSpec(memory_space=pltpu.SMEM),
        ],
        out_specs=pl.BlockSpec((RB // 128, 128), lambda i: (i, 0)),
        out_shape=jax.ShapeDtypeStruct((NPAD // 128, 128), jnp.int32),
    )(node_embs, scorer, mask, s2d)


def _i32c(x):
    x &= 0xFFFFFFFF
    return x - (1 << 32) if x >= (1 << 31) else x


def _lane_sum(v):
    s = v[0]
    for l in range(1, 16):
        s = s + v[l]
    return s


def _sc_body(keys_hbm, embs_hbm, out_hbm,
             keys_v, rdcnt_v, cnt_v, ckl_v, cil_v, gck_v, gci_v, ck_v, ci_v,
             rankidx_v, rowbuf_v, srd_v, sidx_v, rows_v,
             sh_cnt, sh_ck, sh_ci, sh_srt, sem):
    t = lax.axis_index("s")
    lanes = lax.iota(jnp.int32, 16)
    base = t * CH
    z16 = jnp.zeros((16,), jnp.int32)

    pltpu.sync_copy(keys_hbm.at[pl.ds(base, CH)], keys_v)

    bb = jnp.int32(INT_MIN)
    for it in range(16):
        w = 30 - 2 * it
        f1 = jnp.full((16,), bb + jnp.int32(_i32c(1 << w)), jnp.int32)
        f2 = jnp.full((16,), bb + jnp.int32(_i32c(2 << w)), jnp.int32)
        f3 = jnp.full((16,), bb + jnp.int32(_i32c(3 << w)), jnp.int32)

        def scan(c, carry, f1=f1, f2=f2, f3=f3):
            a1, a2, a3 = carry
            k = keys_v[pl.ds(c * 16, 16)]
            a1 = a1 + jnp.where(k >= f1, 1, 0)
            a2 = a2 + jnp.where(k >= f2, 1, 0)
            a3 = a3 + jnp.where(k >= f3, 1, 0)
            return a1, a2, a3
        a1, a2, a3 = lax.fori_loop(0, NCHUNK, scan, (z16, z16, z16))
        cnt_v[pl.ds(0, 16)] = jnp.full((16,), _lane_sum(a1), jnp.int32)
        cnt_v[pl.ds(16, 16)] = jnp.full((16,), _lane_sum(a2), jnp.int32)
        cnt_v[pl.ds(32, 16)] = jnp.full((16,), _lane_sum(a3), jnp.int32)
        buf = (it & 1) * 2048
        pltpu.sync_copy(cnt_v, sh_cnt.at[pl.ds(buf + t * 128, 128)])
        plsc.subcore_barrier()
        pltpu.sync_copy(sh_cnt.at[pl.ds(buf, 2048)], rdcnt_v)

        def comb(u, carry):
            g1, g2, g3 = carry
            g1 = g1 + rdcnt_v[pl.ds(u * 128, 16)]
            g2 = g2 + rdcnt_v[pl.ds(u * 128 + 16, 16)]
            g3 = g3 + rdcnt_v[pl.ds(u * 128 + 32, 16)]
            return g1, g2, g3
        g1, g2, g3 = lax.fori_loop(0, NT, comb, (z16, z16, z16))
        t1, t2, t3 = g1[0], g2[0], g3[0]
        kk = jnp.int32(K)
        bb = bb + jnp.where(t3 >= kk, jnp.int32(_i32c(3 << w)),
                            jnp.where(t2 >= kk, jnp.int32(_i32c(2 << w)),
                                      jnp.where(t1 >= kk, jnp.int32(_i32c(1 << w)),
                                                jnp.int32(0))))
    thresh = bb

    pend0k = jnp.full((16,), INT_MIN, jnp.int32)
    pend0i = jnp.full((16,), 0x7FFFFFFF, jnp.int32)

    def zl(i, _):
        ckl_v[pl.ds(i * 16, 16)] = pend0k
        cil_v[pl.ds(i * 16, 16)] = pend0i
        return 0
    lax.fori_loop(0, CAP_L // 16, zl, 0)

    def cchunk(c, carry):
        pk, pi, pcnt, wb = carry
        kv = keys_v[pl.ds(c * 16, 16)]
        for l in range(16):
            kl = kv[l]
            hit = kl >= thresh
            put = jnp.where(hit, pcnt, jnp.int32(-1)) == lanes
            pk = jnp.where(put, jnp.full((16,), kl, jnp.int32), pk)
            pi = jnp.where(put, jnp.full((16,), base + c * 16 + l, jnp.int32), pi)
            pcnt = pcnt + jnp.where(hit, 1, 0)
            flush = pcnt >= 16
            wbs = jnp.minimum(wb, CAP_L - 16)

            @pl.when(flush)
            def _(pk=pk, pi=pi, wbs=wbs):
                ckl_v[pl.ds(wbs, 16)] = pk
                cil_v[pl.ds(wbs, 16)] = pi
            pk = jnp.where(flush, pend0k, pk)
            pi = jnp.where(flush, pend0i, pi)
            pcnt = jnp.where(flush, 0, pcnt)
            wb = jnp.where(flush, jnp.minimum(wb + 16, CAP_L - 16), wb)
        return pk, pi, pcnt, wb
    pk, pi, pcnt, wb = lax.fori_loop(
        0, NCHUNK, cchunk, (pend0k, pend0i, jnp.int32(0), jnp.int32(0)))
    wbs = jnp.minimum(wb, CAP_L - 16)

    @pl.when(pcnt > 0)
    def _():
        ckl_v[pl.ds(wbs, 16)] = pk
        cil_v[pl.ds(wbs, 16)] = pi

    pltpu.sync_copy(ckl_v, sh_ck.at[pl.ds(t * CAP_L, CAP_L)])
    pltpu.sync_copy(cil_v, sh_ci.at[pl.ds(t * CAP_L, CAP_L)])
    plsc.subcore_barrier()

    pltpu.sync_copy(sh_ck, gck_v)
    pltpu.sync_copy(sh_ci, gci_v)

    def zg(i, _):
        ck_v[pl.ds(i * 16, 16)] = pend0k
        ci_v[pl.ds(i * 16, 16)] = pend0i
        return 0
    lax.fori_loop(0, CAP_G // 16, zg, 0)

    def gchunk(c, carry):
        pk, pi, pcnt, wb = carry
        kv = gck_v[pl.ds(c * 16, 16)]
        iv = gci_v[pl.ds(c * 16, 16)]
        for l in range(16):
            kl = kv[l]
            hit = kl > jnp.int32(INT_MIN)
            put = jnp.where(hit, pcnt, jnp.int32(-1)) == lanes
            pk = jnp.where(put, jnp.full((16,), kl, jnp.int32), pk)
            pi = jnp.where(put, jnp.full((16,), iv[l], jnp.int32), pi)
            pcnt = pcnt + jnp.where(hit, 1, 0)
            flush = pcnt >= 16
            wbs = jnp.minimum(wb, CAP_G - 16)

            @pl.when(flush)
            def _(pk=pk, pi=pi, wbs=wbs):
                ck_v[pl.ds(wbs, 16)] = pk
                ci_v[pl.ds(wbs, 16)] = pi
            pk = jnp.where(flush, pend0k, pk)
            pi = jnp.where(flush, pend0i, pi)
            pcnt = jnp.where(flush, 0, pcnt)
            wb = jnp.where(flush, jnp.minimum(wb + 16, CAP_G - 16), wb)
        return pk, pi, pcnt, wb
    pk, pi, pcnt, wb = lax.fori_loop(
        0, (NT * CAP_L) // 16, gchunk, (pend0k, pend0i, jnp.int32(0), jnp.int32(0)))
    wbs = jnp.minimum(wb, CAP_G - 16)

    @pl.when(pcnt > 0)
    def _():
        ck_v[pl.ds(wbs, 16)] = pk
        ci_v[pl.ds(wbs, 16)] = pi
    n_cand = jnp.minimum(wb + pcnt, jnp.int32(CAP_G))
    jchunks = (n_cand + 15) >> 4

    ngrp = RPT // 16
    kc = [ck_v[pl.ds(t * RPT + g * 16, 16)] for g in range(ngrp)]
    ic = [ci_v[pl.ds(t * RPT + g * 16, 16)] for g in range(ngrp)]

    def jstep(jc, accs):
        kv = ck_v[pl.ds(jc * 16, 16)]
        iv = ci_v[pl.ds(jc * 16, 16)]
        out = list(accs)
        for l in range(16):
            kj = jnp.full((16,), kv[l], jnp.int32)
            ij = jnp.full((16,), iv[l], jnp.int32)
            for g in range(ngrp):
                cond = (kj > kc[g]) | ((kj == kc[g]) & (ij < ic[g]))
                out[g] = out[g] + jnp.where(cond, 1, 0)
        return tuple(out)
    accs = lax.fori_loop(0, jchunks, jstep, tuple(z16 for _ in range(ngrp)))

    for g in range(ngrp):
        rankidx_v[pl.ds(g * 16, 16)] = jnp.minimum(accs[g], jnp.int32(KP - 1))
        iv = ci_v[pl.ds(t * RPT + g * 16, 16)]
        for l in range(16):
            rowbuf_v[g * 16 + l, 0, pl.ds(0, 16)] = jnp.full((16,), iv[l], jnp.int32)

    pltpu.sync_copy(rowbuf_v, sh_srt.at[rankidx_v])
    plsc.subcore_barrier()

    pltpu.sync_copy(sh_srt.at[pl.ds(t * GPT, GPT)], srd_v)
    nmax = jnp.full((16,), N - 1, jnp.int32)
    pendg = z16
    for j in range(GPT):
        v = srd_v[j, 0, pl.ds(0, 16)]
        put = lanes == (j % 16)
        pendg = jnp.where(put, jnp.full((16,), v[0], jnp.int32), pendg)
        if j % 16 == 15:
            pendg = jnp.minimum(jnp.maximum(pendg, 0), nmax)
            sidx_v[pl.ds((j // 16) * 16, 16)] = pendg

    pltpu.async_copy(embs_hbm.at[sidx_v], rows_v, sem).wait()
    pltpu.sync_copy(rows_v, out_hbm.at[pl.ds(t * GPT, GPT)])


def _sc_topk_gather(keys, node_embs):
    mesh = plsc.VectorSubcoreMesh(
        core_axis_name="c", subcore_axis_name="s", num_cores=1)
    fn = pl.kernel(
        _sc_body,
        out_type=jax.ShapeDtypeStruct((KP, FEATS), jnp.float32),
        mesh=mesh,
        scratch_types=[
            pltpu.VMEM((CH,), jnp.int32),
            pltpu.VMEM((2048,), jnp.int32),
            pltpu.VMEM((128,), jnp.int32),
            pltpu.VMEM((CAP_L,), jnp.int32),
            pltpu.VMEM((CAP_L,), jnp.int32),
            pltpu.VMEM((NT * CAP_L,), jnp.int32),
            pltpu.VMEM((NT * CAP_L,), jnp.int32),
            pltpu.VMEM((CAP_G,), jnp.int32),
            pltpu.VMEM((CAP_G,), jnp.int32),
            pltpu.VMEM((RPT,), jnp.int32),
            pltpu.VMEM((RPT, 1, 128), jnp.int32),
            pltpu.VMEM((GPT, 1, 128), jnp.int32),
            pltpu.VMEM((GPT,), jnp.int32),
            pltpu.VMEM((GPT, FEATS), jnp.float32),
            pltpu.VMEM_SHARED((4096,), jnp.int32),
            pltpu.VMEM_SHARED((NT * CAP_L,), jnp.int32),
            pltpu.VMEM_SHARED((NT * CAP_L,), jnp.int32),
            pltpu.VMEM_SHARED((KP, 1, 128), jnp.int32),
            pltpu.SemaphoreType.DMA,
        ],
    )
    return fn(keys, node_embs)


def _transpose_body(x_ref, o_ref):
    o_ref[...] = x_ref[...].T


def _transpose(rows):
    return pl.pallas_call(
        _transpose_body,
        grid=(16,),
        in_specs=[pl.BlockSpec((128, FEATS), lambda i: (i, 0))],
        out_specs=pl.BlockSpec((FEATS, 128), lambda i: (0, i)),
        out_shape=jax.ShapeDtypeStruct((FEATS, 2048), jnp.float32),
    )(rows)


def kernel(node_embs, mask, scorer):
    s2d = jnp.sum(scorer).reshape(1, 1)
    keys = _scores_to_keys(node_embs, mask, scorer, s2d).reshape(-1)
    rows = _sc_topk_gather(keys, node_embs)
    out = _transpose(rows)
    return out[:, :K]
